# baseline (speedup 1.0000x reference)
import jax
import jax.numpy as jnp
from jax import lax
from jax.experimental import pallas as pl
from jax.experimental.pallas import tpu as pltpu
from jax.experimental.pallas import tpu_sc as plsc

BATCH = 16384
EMB = 32
NUM_CORES = 2
NUM_SUBCORES = 16
NUM_WORKERS = NUM_CORES * NUM_SUBCORES
BPW = BATCH // NUM_WORKERS
CHUNK = 128
NCHUNKS = BPW // CHUNK
LANES = 16

_MESH = plsc.VectorSubcoreMesh(core_axis_name="c", subcore_axis_name="s")
_PARAMS = pltpu.CompilerParams(
    needs_layout_passes=False, use_tc_tiling_on_sc=False)


def _gather_body(idx_hbm, tab_hbm, out_hbm, idx, rows, sem):
    wid = lax.axis_index("s") * NUM_CORES + lax.axis_index("c")
    pltpu.sync_copy(idx_hbm.at[wid], idx)
    for c in range(NCHUNKS):
        pltpu.async_copy(
            tab_hbm.at[idx.at[c]], rows.at[pl.ds(c * CHUNK, CHUNK)], sem
        ).wait()
    pltpu.sync_copy(rows, out_hbm.at[pl.ds(wid * BPW, BPW)])


def _dot_body(u_hbm, i_hbm, out_hbm, urows, irows, outv, usem, isem):
    wid = lax.axis_index("s") * NUM_CORES + lax.axis_index("c")
    ucp = pltpu.async_copy(u_hbm.at[pl.ds(wid * BPW, BPW)], urows, usem)
    icp = pltpu.async_copy(i_hbm.at[pl.ds(wid * BPW, BPW)], irows, isem)
    ucp.wait()
    icp.wait()

    @pl.loop(0, BPW, step=LANES)
    def _(i0):
        rows = lax.iota(jnp.int32, LANES) + i0
        cols0 = jnp.zeros((LANES,), jnp.int32)
        acc = (plsc.load_gather(urows, [rows, cols0]) *
               plsc.load_gather(irows, [rows, cols0]))
        for j in range(1, EMB):
            cols = jnp.full((LANES,), j, jnp.int32)
            acc = acc + (plsc.load_gather(urows, [rows, cols]) *
                         plsc.load_gather(irows, [rows, cols]))
        outv[pl.ds(i0, LANES)] = acc

    pltpu.sync_copy(outv, out_hbm.at[pl.ds(wid * BPW, BPW)])


def _gather(idx3, table):
    run = pl.kernel(
        _gather_body,
        out_type=jax.ShapeDtypeStruct((BATCH, EMB), jnp.float32),
        mesh=_MESH,
        compiler_params=_PARAMS,
        scratch_types=[
            pltpu.VMEM((NCHUNKS, CHUNK), jnp.int32),
            pltpu.VMEM((BPW, EMB), jnp.float32),
            pltpu.SemaphoreType.DMA,
        ],
    )
    return run(idx3, table)


@jax.jit
def _mf(user3, item3, user_table, item_table):
    u_rows = _gather(user3, user_table)
    i_rows = _gather(item3, item_table)
    run = pl.kernel(
        _dot_body,
        out_type=jax.ShapeDtypeStruct((BATCH,), jnp.float32),
        mesh=_MESH,
        compiler_params=_PARAMS,
        scratch_types=[
            pltpu.VMEM((BPW, EMB), jnp.float32),
            pltpu.VMEM((BPW, EMB), jnp.float32),
            pltpu.VMEM((BPW,), jnp.float32),
            pltpu.SemaphoreType.DMA,
            pltpu.SemaphoreType.DMA,
        ],
    )
    return run(u_rows, i_rows)


def kernel(user, item, user_table, item_table):
    user3 = user.astype(jnp.int32).reshape(NUM_WORKERS, NCHUNKS, CHUNK)
    item3 = item.astype(jnp.int32).reshape(NUM_WORKERS, NCHUNKS, CHUNK)
    return _mf(user3, item3, user_table, item_table)
